# Initial kernel scaffold; baseline (speedup 1.0000x reference)
import jax
import jax.numpy as jnp
from jax.experimental import pallas as pl


def kernel(input_ids, W):
    raise NotImplementedError("write your pallas kernel here")


# speedup vs baseline: 3.3717x; 3.3717x over previous
import functools
import math

import jax
import jax.numpy as jnp
from jax import lax
from jax.experimental import pallas as pl
from jax.experimental.pallas import tpu as pltpu
from jax.experimental.pallas import tpu_sc as plsc

_VOCAB = 100000
_HID = 2048
_HEAD_DIM = 128
_BASE = 10000.0
_B, _S = 2, 2048

_NTOK = _B * _S
_NW = 32
_BPW = _NTOK // _NW
_CH = 16
_NCH = _BPW // _CH


@functools.partial(
    pl.kernel,
    out_type=jax.ShapeDtypeStruct((_NTOK, _HID), jnp.float32),
    mesh=plsc.VectorSubcoreMesh(core_axis_name="c", subcore_axis_name="s"),
    scratch_types=[
        pltpu.VMEM((_NCH, _CH), jnp.int32),
        pltpu.VMEM((_CH, _HID), jnp.float32),
        pltpu.VMEM((_CH, _HID), jnp.float32),
        pltpu.SemaphoreType.DMA,
        pltpu.SemaphoreType.DMA,
    ],
)
def _emb_gather(ids_hbm, w_hbm, out_hbm, idx_v, buf0, buf1, sem0, sem1):
    wid = lax.axis_index("s") * 2 + lax.axis_index("c")
    pltpu.sync_copy(ids_hbm.at[wid], idx_v)

    bufs = (buf0, buf1)
    sems = (sem0, sem1)

    def gather(k):
        b = k % 2
        return pltpu.make_async_copy(w_hbm.at[idx_v.at[k]], bufs[b], sems[b])

    gather(0).start()
    gather(1).start()
    for k in range(_NCH):
        gather(k).wait()
        pltpu.sync_copy(bufs[k % 2], out_hbm.at[pl.ds(wid * _BPW + k * _CH, _CH)])
        if k + 2 < _NCH:
            gather(k + 2).start()


def _rot_body(cos_ref, sin_ref):
    shape = (1, 1, _S, _HEAD_DIM)
    pos = lax.broadcasted_iota(jnp.int32, shape, 2).astype(jnp.float32)
    col = lax.broadcasted_iota(jnp.int32, shape, 3)
    half = _HEAD_DIM // 2
    j = jnp.where(col < half, col, col - half).astype(jnp.float32)
    inv_freq = jnp.exp(j * (-math.log(_BASE) / half))
    freqs = pos * inv_freq
    cos_ref[...] = jnp.cos(freqs)
    sin_ref[...] = jnp.sin(freqs)


_rot = pl.pallas_call(
    _rot_body,
    grid=(_B,),
    out_shape=[jax.ShapeDtypeStruct((_B, 1, _S, _HEAD_DIM), jnp.float32)] * 2,
    out_specs=[pl.BlockSpec((1, 1, _S, _HEAD_DIM), lambda b: (b, 0, 0, 0))] * 2,
)


def kernel(input_ids, W):
    ids3 = input_ids.reshape(_NW, _NCH, _CH)
    flat = _emb_gather(ids3, W)
    hidden_states = flat.reshape(_B, _S, _HID)
    cos_g, sin_g = _rot()
    return (hidden_states, cos_g, sin_g)
